# baseline (speedup 1.0000x reference)
import functools

import jax
import jax.numpy as jnp
from jax import lax
from jax.experimental import pallas as pl
from jax.experimental.pallas import tpu as pltpu
from jax.experimental.pallas import tpu_sc as plsc

NUM_FREQ = 4
IN_DIM = 4
OUT_DIM = 4
NUM_BINS = 34
ROW = OUT_DIM * IN_DIM * NUM_FREQ
E = 800000
ETILES = E // 128

NC = 2
NS = 16
NW = NC * NS
L = 16

ST = 5
EPB = ST * 128
NSUP = ETILES // ST
NIT = -(-NSUP // NW)

_MAGIC = 8388608.0


def _bins_from_dists(d):
    x = jnp.clip((d - 2.4) / 0.4, 0.0, 33.0)
    r = (x + _MAGIC) - _MAGIC
    return r.astype(jnp.int32)


_mesh = plsc.VectorSubcoreMesh(core_axis_name="c", subcore_axis_name="s")


@functools.partial(
    pl.kernel,
    mesh=_mesh,
    out_type=jax.ShapeDtypeStruct((ROW // NUM_FREQ, NUM_FREQ, ETILES, 128),
                                  jnp.float32),
    scratch_types=[
        pltpu.VMEM((NUM_BINS * (ROW + 1),), jnp.float32),
        [pltpu.VMEM((EPB,), jnp.float32) for _ in range(2)],
        [pltpu.VMEM((EPB,), jnp.int32) for _ in range(2)],
        [pltpu.VMEM((ROW // NUM_FREQ, NUM_FREQ, ST, 128), jnp.float32)
         for _ in range(2)],
        [pltpu.SemaphoreType.DMA for _ in range(2)],
        [pltpu.SemaphoreType.DMA for _ in range(2)],
    ],
    compiler_params=pltpu.CompilerParams(use_tc_tiling_on_sc=False,
                                         needs_layout_passes=False),
)
def _radial_sc(dists_hbm, table_hbm, out_hbm, tbl_v, d_v, bin_v, tbuf, sem_w,
               sem_d):
    wid = lax.axis_index("s") * NC + lax.axis_index("c")

    pltpu.sync_copy(table_hbm, tbl_v)

    def prefetch(s, b):
        pltpu.async_copy(dists_hbm.at[pl.ds(s * EPB, EPB)], d_v[b], sem_d[b])

    prefetch(wid, 0)
    prefetch(wid + NW, 1)

    def drain(b):
        pltpu.make_async_copy(
            tbuf[b], out_hbm.at[:, :, pl.ds(0, ST)], sem_w[b]).wait()

    def process(s, b):
        pltpu.make_async_copy(dists_hbm.at[pl.ds(0, EPB)], d_v[b],
                              sem_d[b]).wait()

        @pl.loop(0, EPB // L)
        def _(g):
            bin_v[b][pl.ds(g * L, L)] = _bins_from_dists(d_v[b][pl.ds(g * L, L)])

        @pl.when(s + 2 * NW < NSUP)
        def _():
            prefetch(s + 2 * NW, b)

        @pl.loop(0, ST)
        def _(t):
            @pl.loop(0, 128 // L)
            def _(eg):
                base = bin_v[b][pl.ds(t * 128 + eg * L, L)] * (ROW + 1)
                idxs = [base + f for f in range(NUM_FREQ)]
                for oi in range(ROW // NUM_FREQ):
                    for f in range(NUM_FREQ):
                        v = plsc.load_gather(tbl_v, [idxs[f]])
                        tbuf[b][oi, f, t, pl.ds(eg * L, L)] = v
                    if oi < ROW // NUM_FREQ - 1:
                        idxs = [i + NUM_FREQ for i in idxs]

        pltpu.async_copy(tbuf[b], out_hbm.at[:, :, pl.ds(s * ST, ST)],
                         sem_w[b])

    @pl.loop(0, NIT, step=2)
    def _(j):
        for b in range(2):
            k = j + b
            s = wid + k * NW

            @pl.when(k >= 2)
            def _():
                drain(b)

            @pl.when(s < NSUP)
            def _():
                process(s, b)

    drain((NIT - 2) % 2)

    @pl.when(wid < NSUP - (NIT - 1) * NW)
    def _():
        drain((NIT - 1) % 2)


def kernel(dists, bin_embedding):
    tpad = jnp.pad(bin_embedding, ((0, 0), (0, 1))).reshape(NUM_BINS * (ROW + 1))
    x = _radial_sc(dists.reshape(E), tpad)
    x = x.reshape(OUT_DIM, IN_DIM, NUM_FREQ, ETILES, 128)
    x = x.transpose(3, 4, 0, 1, 2).reshape(E, OUT_DIM, IN_DIM, NUM_FREQ)
    return x[:, :, None, :, None, :]


# speedup vs baseline: 1.0060x; 1.0060x over previous
import functools

import jax
import jax.numpy as jnp
from jax import lax
from jax.experimental import pallas as pl
from jax.experimental.pallas import tpu as pltpu
from jax.experimental.pallas import tpu_sc as plsc

NUM_FREQ = 4
IN_DIM = 4
OUT_DIM = 4
NUM_BINS = 34
ROW = OUT_DIM * IN_DIM * NUM_FREQ
E = 800000
ETILES = E // 128

NC = 2
NS = 16
NW = NC * NS
L = 16

ST = 5
EPB = ST * 128
NSUP = ETILES // ST
NIT = -(-NSUP // NW)

_MAGIC = 8388608.0


def _bins_from_dists(d):
    x = jnp.clip((d - 2.4) / 0.4, 0.0, 33.0)
    r = (x + _MAGIC) - _MAGIC
    return r.astype(jnp.int32)


_mesh = plsc.VectorSubcoreMesh(core_axis_name="c", subcore_axis_name="s")


@functools.partial(
    pl.kernel,
    mesh=_mesh,
    out_type=jax.ShapeDtypeStruct((ROW // NUM_FREQ, ETILES, NUM_FREQ, 128),
                                  jnp.float32),
    scratch_types=[
        pltpu.VMEM((NUM_BINS * (ROW + 1),), jnp.float32),
        [pltpu.VMEM((EPB,), jnp.float32) for _ in range(2)],
        [pltpu.VMEM((EPB,), jnp.int32) for _ in range(2)],
        [pltpu.VMEM((ROW // NUM_FREQ, ST, NUM_FREQ, 128), jnp.float32)
         for _ in range(2)],
        [pltpu.SemaphoreType.DMA for _ in range(2)],
        [pltpu.SemaphoreType.DMA for _ in range(2)],
    ],
    compiler_params=pltpu.CompilerParams(use_tc_tiling_on_sc=False,
                                         needs_layout_passes=False),
)
def _radial_sc(dists_hbm, table_hbm, out_hbm, tbl_v, d_v, bin_v, tbuf, sem_w,
               sem_d):
    wid = lax.axis_index("s") * NC + lax.axis_index("c")

    pltpu.sync_copy(table_hbm, tbl_v)

    def prefetch(s, b):
        pltpu.async_copy(dists_hbm.at[pl.ds(s * EPB, EPB)], d_v[b], sem_d[b])

    prefetch(wid, 0)
    prefetch(wid + NW, 1)

    def drain(b):
        pltpu.make_async_copy(
            tbuf[b], out_hbm.at[:, pl.ds(0, ST)], sem_w[b]).wait()

    def process(s, b):
        pltpu.make_async_copy(dists_hbm.at[pl.ds(0, EPB)], d_v[b],
                              sem_d[b]).wait()

        @pl.loop(0, EPB // L)
        def _(g):
            bin_v[b][pl.ds(g * L, L)] = _bins_from_dists(d_v[b][pl.ds(g * L, L)])

        @pl.when(s + 2 * NW < NSUP)
        def _():
            prefetch(s + 2 * NW, b)

        @pl.loop(0, ST)
        def _(t):
            @pl.loop(0, 128 // L)
            def _(eg):
                base = bin_v[b][pl.ds(t * 128 + eg * L, L)] * (ROW + 1)
                idxs = [base + f for f in range(NUM_FREQ)]
                for oi in range(ROW // NUM_FREQ):
                    for f in range(NUM_FREQ):
                        v = plsc.load_gather(tbl_v, [idxs[f]])
                        tbuf[b][oi, t, f, pl.ds(eg * L, L)] = v
                    if oi < ROW // NUM_FREQ - 1:
                        idxs = [i + NUM_FREQ for i in idxs]

        pltpu.async_copy(tbuf[b], out_hbm.at[:, pl.ds(s * ST, ST)], sem_w[b])

    @pl.loop(0, NIT, step=2)
    def _(j):
        for b in range(2):
            k = j + b
            s = wid + k * NW

            @pl.when(k >= 2)
            def _():
                drain(b)

            @pl.when(s < NSUP)
            def _():
                process(s, b)

    drain((NIT - 2) % 2)

    @pl.when(wid < NSUP - (NIT - 1) * NW)
    def _():
        drain((NIT - 1) % 2)


def kernel(dists, bin_embedding):
    tpad = jnp.pad(bin_embedding, ((0, 0), (0, 1))).reshape(NUM_BINS * (ROW + 1))
    x = _radial_sc(dists.reshape(E), tpad)
    x = x.reshape(OUT_DIM, IN_DIM, ETILES, NUM_FREQ, 128)
    x = x.transpose(2, 4, 0, 1, 3).reshape(E, OUT_DIM, IN_DIM, NUM_FREQ)
    return x[:, :, None, :, None, :]


# speedup vs baseline: 1.0100x; 1.0039x over previous
import functools

import jax
import jax.numpy as jnp
from jax import lax
from jax.experimental import pallas as pl
from jax.experimental.pallas import tpu as pltpu
from jax.experimental.pallas import tpu_sc as plsc

NUM_FREQ = 4
IN_DIM = 4
OUT_DIM = 4
NUM_BINS = 34
ROW = OUT_DIM * IN_DIM * NUM_FREQ
E = 800000
ETILES = E // 128

NC = 2
NS = 16
NW = NC * NS
L = 16

ST = 5
EPB = ST * 128
NSUP = ETILES // ST
NIT = -(-NSUP // NW)

_MAGIC = 8388608.0


def _bins_from_dists(d):
    x = jnp.clip((d - 2.4) / 0.4, 0.0, 33.0)
    r = (x + _MAGIC) - _MAGIC
    return r.astype(jnp.int32)


_mesh = plsc.VectorSubcoreMesh(core_axis_name="c", subcore_axis_name="s")


@functools.partial(
    pl.kernel,
    mesh=_mesh,
    out_type=jax.ShapeDtypeStruct((ROW // NUM_FREQ, ETILES, NUM_FREQ, 128),
                                  jnp.float32),
    scratch_types=[
        pltpu.VMEM((NUM_BINS * (ROW + 1),), jnp.float32),
        [pltpu.VMEM((EPB,), jnp.float32) for _ in range(2)],
        [pltpu.VMEM((EPB,), jnp.int32) for _ in range(2)],
        [pltpu.VMEM((ROW // NUM_FREQ, ST, NUM_FREQ, 128), jnp.float32)
         for _ in range(2)],
        [pltpu.SemaphoreType.DMA for _ in range(2)],
        [pltpu.SemaphoreType.DMA for _ in range(2)],
    ],
    compiler_params=pltpu.CompilerParams(use_tc_tiling_on_sc=False,
                                         needs_layout_passes=False),
)
def _radial_sc(dists_hbm, table_hbm, out_hbm, tbl_v, d_v, bin_v, tbuf, sem_w,
               sem_d):
    wid = lax.axis_index("s") * NC + lax.axis_index("c")

    pltpu.sync_copy(table_hbm, tbl_v)

    def prefetch(s, b):
        pltpu.async_copy(dists_hbm.at[pl.ds(s * EPB, EPB)], d_v[b], sem_d[b])

    prefetch(wid, 0)
    prefetch(wid + NW, 1)

    def drain(b):
        pltpu.make_async_copy(
            tbuf[b], out_hbm.at[:, pl.ds(0, ST)], sem_w[b]).wait()

    def process(s, b):
        pltpu.make_async_copy(dists_hbm.at[pl.ds(0, EPB)], d_v[b],
                              sem_d[b]).wait()

        @pl.loop(0, EPB // L)
        def _(g):
            bin_v[b][pl.ds(g * L, L)] = _bins_from_dists(d_v[b][pl.ds(g * L, L)])

        @pl.when(s + 2 * NW < NSUP)
        def _():
            prefetch(s + 2 * NW, b)

        @pl.loop(0, ST)
        def _(t):
            @pl.loop(0, 128 // L // 2)
            def _(eh):
                bases = [
                    bin_v[b][pl.ds(t * 128 + (2 * eh + h) * L, L)] * (ROW + 1)
                    for h in range(2)
                ]
                idxs = [[base + f for f in range(NUM_FREQ)] for base in bases]
                for oi in range(ROW // NUM_FREQ):
                    for h in range(2):
                        for f in range(NUM_FREQ):
                            v = plsc.load_gather(tbl_v, [idxs[h][f]])
                            tbuf[b][oi, t, f, pl.ds((2 * eh + h) * L, L)] = v
                    if oi < ROW // NUM_FREQ - 1:
                        idxs = [[i + NUM_FREQ for i in row] for row in idxs]

        pltpu.async_copy(tbuf[b], out_hbm.at[:, pl.ds(s * ST, ST)], sem_w[b])

    @pl.loop(0, NIT, step=2)
    def _(j):
        for b in range(2):
            k = j + b
            s = wid + k * NW

            @pl.when(k >= 2)
            def _():
                drain(b)

            @pl.when(s < NSUP)
            def _():
                process(s, b)

    drain((NIT - 2) % 2)

    @pl.when(wid < NSUP - (NIT - 1) * NW)
    def _():
        drain((NIT - 1) % 2)


def kernel(dists, bin_embedding):
    tpad = jnp.pad(bin_embedding, ((0, 0), (0, 1))).reshape(NUM_BINS * (ROW + 1))
    x = _radial_sc(dists.reshape(E), tpad)
    x = x.reshape(OUT_DIM, IN_DIM, ETILES, NUM_FREQ, 128)
    x = x.transpose(2, 4, 0, 1, 3).reshape(E, OUT_DIM, IN_DIM, NUM_FREQ)
    return x[:, :, None, :, None, :]


# speedup vs baseline: 1.0739x; 1.0632x over previous
import functools

import jax
import jax.numpy as jnp
from jax import lax
from jax.experimental import pallas as pl
from jax.experimental.pallas import tpu as pltpu
from jax.experimental.pallas import tpu_sc as plsc

NUM_FREQ = 4
IN_DIM = 4
OUT_DIM = 4
NUM_BINS = 34
ROW = OUT_DIM * IN_DIM * NUM_FREQ
E = 800000
ETILES = E // 128

NC = 2
NS = 16
NW = NC * NS
L = 16

ST = 5
EPB = ST * 128
NSUP = ETILES // ST
NIT = -(-NSUP // NW)

_MAGIC = 8388608.0


def _bins_from_dists(d):
    x = jnp.clip((d - 2.4) / 0.4, 0.0, 33.0)
    r = (x + _MAGIC) - _MAGIC
    return r.astype(jnp.int32)


_mesh = plsc.VectorSubcoreMesh(core_axis_name="c", subcore_axis_name="s")


@functools.partial(
    pl.kernel,
    mesh=_mesh,
    out_type=jax.ShapeDtypeStruct((ROW // NUM_FREQ, ETILES, NUM_FREQ, 128),
                                  jnp.float32),
    scratch_types=[
        pltpu.VMEM((L * (NUM_BINS * 80 + 1),), jnp.float32),
        [pltpu.VMEM((EPB,), jnp.float32) for _ in range(2)],
        [pltpu.VMEM((EPB,), jnp.int32) for _ in range(2)],
        [pltpu.VMEM((ROW // NUM_FREQ, ST, NUM_FREQ, 128), jnp.float32)
         for _ in range(2)],
        [pltpu.SemaphoreType.DMA for _ in range(2)],
        [pltpu.SemaphoreType.DMA for _ in range(2)],
    ],
    compiler_params=pltpu.CompilerParams(use_tc_tiling_on_sc=False,
                                         needs_layout_passes=False),
)
def _radial_sc(dists_hbm, table_hbm, out_hbm, tbl_v, d_v, bin_v, tbuf, sem_w,
               sem_d):
    wid = lax.axis_index("s") * NC + lax.axis_index("c")

    pltpu.sync_copy(table_hbm, tbl_v)
    lane_off = lax.iota(jnp.int32, L) * (NUM_BINS * 80 + 1)

    def prefetch(s, b):
        pltpu.async_copy(dists_hbm.at[pl.ds(s * EPB, EPB)], d_v[b], sem_d[b])

    prefetch(wid, 0)
    prefetch(wid + NW, 1)

    def drain(b):
        pltpu.make_async_copy(
            tbuf[b], out_hbm.at[:, pl.ds(0, ST)], sem_w[b]).wait()

    def process(s, b):
        pltpu.make_async_copy(dists_hbm.at[pl.ds(0, EPB)], d_v[b],
                              sem_d[b]).wait()

        @pl.loop(0, EPB // L)
        def _(g):
            bin_v[b][pl.ds(g * L, L)] = _bins_from_dists(d_v[b][pl.ds(g * L, L)])

        @pl.when(s + 2 * NW < NSUP)
        def _():
            prefetch(s + 2 * NW, b)

        @pl.loop(0, ST)
        def _(t):
            @pl.loop(0, 128 // L // 2)
            def _(eh):
                bases = [
                    bin_v[b][pl.ds(t * 128 + (2 * eh + h) * L, L)] * 80
                    + lane_off
                    for h in range(2)
                ]
                idxs = [[base + f for f in range(NUM_FREQ)] for base in bases]
                for oi in range(ROW // NUM_FREQ):
                    for h in range(2):
                        for f in range(NUM_FREQ):
                            v = plsc.load_gather(tbl_v, [idxs[h][f]])
                            tbuf[b][oi, t, f, pl.ds((2 * eh + h) * L, L)] = v
                    if oi < ROW // NUM_FREQ - 1:
                        idxs = [[i + NUM_FREQ for i in row] for row in idxs]

        pltpu.async_copy(tbuf[b], out_hbm.at[:, pl.ds(s * ST, ST)], sem_w[b])

    @pl.loop(0, NIT, step=2)
    def _(j):
        for b in range(2):
            k = j + b
            s = wid + k * NW

            @pl.when(k >= 2)
            def _():
                drain(b)

            @pl.when(s < NSUP)
            def _():
                process(s, b)

    drain((NIT - 2) % 2)

    @pl.when(wid < NSUP - (NIT - 1) * NW)
    def _():
        drain((NIT - 1) % 2)


def kernel(dists, bin_embedding):
    t80 = jnp.pad(bin_embedding, ((0, 0), (0, 16))).reshape(NUM_BINS * 80)
    rep = jnp.tile(jnp.pad(t80, (0, 1)), L)
    x = _radial_sc(dists.reshape(E), rep)
    x = x.reshape(OUT_DIM, IN_DIM, ETILES, NUM_FREQ, 128)
    x = x.transpose(2, 4, 0, 1, 3).reshape(E, OUT_DIM, IN_DIM, NUM_FREQ)
    return x[:, :, None, :, None, :]
